# baseline (speedup 1.0000x reference)
import functools

import jax
import jax.numpy as jnp
from jax import lax
from jax.experimental import pallas as pl
from jax.experimental.pallas import tpu as pltpu
from jax.experimental.pallas import tpu_sc as plsc

_NC = 2
_NS = 16
_NW = _NC * _NS


def _sc_gather_rows(scales, shifts, labels):
    B = labels.shape[0]
    V, D = scales.shape
    b_per_w = B // _NW
    mesh = plsc.VectorSubcoreMesh(core_axis_name="c", subcore_axis_name="s")

    h = b_per_w // 2

    @functools.partial(
        pl.kernel,
        mesh=mesh,
        out_type=jax.ShapeDtypeStruct((2, B, D), scales.dtype),
        scratch_types=[
            pltpu.VMEM((b_per_w,), jnp.int32),
            pltpu.VMEM((h, D), scales.dtype),
            pltpu.VMEM((h, D), scales.dtype),
            pltpu.VMEM((h, D), shifts.dtype),
            pltpu.VMEM((h, D), shifts.dtype),
            pltpu.SemaphoreType.DMA,
            pltpu.SemaphoreType.DMA,
            pltpu.SemaphoreType.DMA,
            pltpu.SemaphoreType.DMA,
            pltpu.SemaphoreType.DMA,
        ],
    )
    def gather_kernel(scales_hbm, shifts_hbm, idx_hbm, rows_out,
                      idx_v, s0_v, s1_v, h0_v, h1_v,
                      sem_s0, sem_s1, sem_h0, sem_h1, sem_wb):
        wid = lax.axis_index("s") * _NC + lax.axis_index("c")
        base = wid * b_per_w
        pltpu.sync_copy(idx_hbm.at[pl.ds(base, b_per_w)], idx_v)
        gs0 = pltpu.async_copy(scales_hbm.at[idx_v.at[pl.ds(0, h)]], s0_v, sem_s0)
        gh0 = pltpu.async_copy(shifts_hbm.at[idx_v.at[pl.ds(0, h)]], h0_v, sem_h0)
        gs1 = pltpu.async_copy(scales_hbm.at[idx_v.at[pl.ds(h, h)]], s1_v, sem_s1)
        gh1 = pltpu.async_copy(shifts_hbm.at[idx_v.at[pl.ds(h, h)]], h1_v, sem_h1)
        gs0.wait()
        w0 = pltpu.async_copy(s0_v, rows_out.at[0, pl.ds(base, h)], sem_wb)
        gh0.wait()
        w1 = pltpu.async_copy(h0_v, rows_out.at[1, pl.ds(base, h)], sem_wb)
        gs1.wait()
        w2 = pltpu.async_copy(s1_v, rows_out.at[0, pl.ds(base + h, h)], sem_wb)
        gh1.wait()
        w3 = pltpu.async_copy(h1_v, rows_out.at[1, pl.ds(base + h, h)], sem_wb)
        w0.wait()
        w1.wait()
        w2.wait()
        w3.wait()

    return gather_kernel(scales, shifts, labels)


def _tc_affine(x, rows):
    S, B, W = x.shape
    blk_b = 2048

    def body(x_ref, r_ref, o_ref):
        o_ref[...] = (x_ref[...] * r_ref[0][None, :, :]
                      + r_ref[1][None, :, :])

    return pl.pallas_call(
        body,
        grid=(B // blk_b,),
        in_specs=[
            pl.BlockSpec((S, blk_b, W), lambda i: (0, i, 0)),
            pl.BlockSpec((2, blk_b, W), lambda i: (0, i, 0)),
        ],
        out_specs=pl.BlockSpec((S, blk_b, W), lambda i: (0, i, 0)),
        out_shape=jax.ShapeDtypeStruct((S, B, W), x.dtype),
    )(x, rows)


def kernel(x, task_labels, num_samples, scales, shifts):
    del num_samples
    labels = task_labels.astype(jnp.int32)
    rows = _sc_gather_rows(scales, shifts, labels)
    return _tc_affine(x, rows)


# speedup vs baseline: 1.0014x; 1.0014x over previous
import functools

import jax
import jax.numpy as jnp
from jax import lax
from jax.experimental import pallas as pl
from jax.experimental.pallas import tpu as pltpu
from jax.experimental.pallas import tpu_sc as plsc

_NC = 2
_NS = 16
_NW = _NC * _NS


def _sc_gather_rows(scales, shifts, labels):
    B = labels.shape[0]
    V, D = scales.shape
    b_per_w = B // _NW
    mesh = plsc.VectorSubcoreMesh(core_axis_name="c", subcore_axis_name="s")

    @functools.partial(
        pl.kernel,
        mesh=mesh,
        out_type=jax.ShapeDtypeStruct((2, B, D), scales.dtype),
        scratch_types=[
            pltpu.VMEM((b_per_w,), jnp.int32),
            pltpu.VMEM((b_per_w, D), scales.dtype),
            pltpu.VMEM((b_per_w, D), shifts.dtype),
            pltpu.SemaphoreType.DMA,
            pltpu.SemaphoreType.DMA,
            pltpu.SemaphoreType.DMA,
            pltpu.SemaphoreType.DMA,
        ],
    )
    def gather_kernel(scales_hbm, shifts_hbm, idx_hbm, rows_out,
                      idx_v, srows_v, hrows_v, sem_a, sem_b, sem_c, sem_d):
        wid = lax.axis_index("s") * _NC + lax.axis_index("c")
        base = wid * b_per_w
        pltpu.sync_copy(idx_hbm.at[pl.ds(base, b_per_w)], idx_v)
        ca = pltpu.async_copy(scales_hbm.at[idx_v], srows_v, sem_a)
        cb = pltpu.async_copy(shifts_hbm.at[idx_v], hrows_v, sem_b)
        ca.wait()
        cc = pltpu.async_copy(srows_v, rows_out.at[0, pl.ds(base, b_per_w)], sem_c)
        cb.wait()
        cd = pltpu.async_copy(hrows_v, rows_out.at[1, pl.ds(base, b_per_w)], sem_d)
        cc.wait()
        cd.wait()

    return gather_kernel(scales, shifts, labels)


def _tc_affine(x, rows):
    S, B, W = x.shape
    blk_b = 2048

    def body(x_ref, r_ref, o_ref):
        o_ref[...] = (x_ref[...] * r_ref[0][None, :, :]
                      + r_ref[1][None, :, :])

    return pl.pallas_call(
        body,
        grid=(B // blk_b,),
        in_specs=[
            pl.BlockSpec((S, blk_b, W), lambda i: (0, i, 0)),
            pl.BlockSpec((2, blk_b, W), lambda i: (0, i, 0)),
        ],
        out_specs=pl.BlockSpec((S, blk_b, W), lambda i: (0, i, 0)),
        out_shape=jax.ShapeDtypeStruct((S, B, W), x.dtype),
    )(x, rows)


def kernel(x, task_labels, num_samples, scales, shifts):
    del num_samples
    labels = task_labels.astype(jnp.int32)
    rows = _sc_gather_rows(scales, shifts, labels)
    return _tc_affine(x, rows)
